# baseline (speedup 1.0000x reference)
import functools

import jax
import jax.numpy as jnp
from jax import lax
from jax.experimental import pallas as pl
from jax.experimental.pallas import tpu as pltpu
from jax.experimental.pallas import tpu_sc as plsc

N_NODES = 10000
D_FEAT = 256
D_EDGE = 16

NC, NS, LANES = 2, 16, 16
NW = NC * NS
K = 128
E_PAD = 163840
GATHER_ITERS = E_PAD // (NW * K)
E_PER_TILE = E_PAD // NS
SCATTER_ITERS = E_PER_TILE // K
N_PAD = 10240

_MESH = plsc.VectorSubcoreMesh(core_axis_name="c", subcore_axis_name="s")
NB = 3
NB_S = 2
CHUNK = GATHER_ITERS * K


PAIR_ITERS = 2 * GATHER_ITERS
IT0 = 25
ITMAX = max(IT0, PAIR_ITERS - IT0)
PAIR_E = PAIR_ITERS * K


@functools.partial(
    pl.kernel,
    mesh=_MESH,
    out_type=jax.ShapeDtypeStruct((E_PAD, D_FEAT), jnp.float32),
    scratch_types=[
        pltpu.VMEM((PAIR_E,), jnp.int32),
        pltpu.VMEM((NB, K, D_FEAT), jnp.float32),
        pltpu.SemaphoreType.DMA((NB,)),
        pltpu.SemaphoreType.DMA((NB,)),
    ],
)
def _sc_gather(x_hbm, col_hbm, g_hbm, idx_v, rows_v, sem_g, sem_s):
    c = lax.axis_index("c")
    s = lax.axis_index("s")
    pair0 = pl.multiple_of(s * PAIR_E, K)
    ib = pl.multiple_of(c * (IT0 * K), K)
    base0 = pl.multiple_of(pair0 + ib, K)
    iters = jnp.where(c == 0, IT0, PAIR_ITERS - IT0)
    pltpu.sync_copy(col_hbm.at[pl.ds(pair0, PAIR_E)], idx_v)

    def gather_desc(i, b):
        off = pl.multiple_of(ib + i * K, K)
        return pltpu.make_async_copy(
            x_hbm.at[idx_v.at[pl.ds(off, K)]], rows_v.at[b], sem_g.at[b])

    def store_desc(i, b):
        off = pl.multiple_of(base0 + i * K, K)
        return pltpu.make_async_copy(
            rows_v.at[b], g_hbm.at[pl.ds(off, K)], sem_s.at[b])

    gather_desc(0, 0).start()

    def body(i, carry):
        b = lax.rem(i, NB)
        nxt = i + 1

        @pl.when(nxt < iters)
        def _():
            bn = lax.rem(nxt, NB)

            @pl.when(nxt >= NB)
            def _():
                store_desc(nxt - NB, bn).wait()

            gather_desc(nxt, bn).start()

        @pl.when(i < iters)
        def _():
            gather_desc(i, b).wait()
            store_desc(i, b).start()
        return carry

    lax.fori_loop(0, ITMAX, body, 0)
    for j in range(NB):
        i = iters - NB + j
        store_desc(i, lax.rem(i, NB)).wait()


ROWS_PER_TILE = N_PAD // NS


@functools.partial(
    pl.kernel,
    mesh=_MESH,
    out_type=jax.ShapeDtypeStruct((NC, N_PAD, 2, K), jnp.float32),
    scratch_types=[
        pltpu.VMEM_SHARED((N_PAD, K), jnp.float32),
        pltpu.VMEM((SCATTER_ITERS, K), jnp.int32),
        pltpu.VMEM((NB_S, K, K), jnp.float32),
        pltpu.SemaphoreType.DMA((NB_S,)),
        pltpu.SemaphoreType.DMA((NB_S,)),
    ],
)
def _sc_scatter(hA, hB, tsel_hbm, agg_hbm,
                acc_sh, idx_v, buf_v, sem_l, sem_sc):
    c = lax.axis_index("c")
    s = lax.axis_index("s")
    my_rows = pl.multiple_of(s * ROWS_PER_TILE, K)
    ebase0 = pl.multiple_of(s * E_PER_TILE, K)
    zeros16 = jnp.zeros((LANES,), jnp.float32)

    pltpu.sync_copy(tsel_hbm.at[c, s], idx_v)

    for ch, h_hbm in ((0, hA), (1, hB)):
        def load_desc(i, b, h_hbm=h_hbm):
            off = pl.multiple_of(ebase0 + i * K, K)
            return pltpu.make_async_copy(
                h_hbm.at[pl.ds(off, K)], buf_v.at[b], sem_l.at[b])

        def scat_desc(i, b):
            return pltpu.make_async_copy(
                buf_v.at[b], acc_sh.at[idx_v.at[i]], sem_sc.at[b])

        def zbody(r, carry):
            for j in range(K // LANES):
                buf_v[0, r, pl.ds(j * LANES, LANES)] = zeros16
            return carry

        lax.fori_loop(0, K, zbody, 0)
        for kk in range(ROWS_PER_TILE // K):
            pltpu.sync_copy(buf_v.at[0],
                            acc_sh.at[pl.ds(my_rows + kk * K, K)])
        plsc.subcore_barrier()

        load_desc(0, 0).start()

        def body(i, carry):
            b = lax.rem(i, NB_S)
            nxt = i + 1

            @pl.when(nxt < SCATTER_ITERS)
            def _():
                bn = lax.rem(nxt, NB_S)

                @pl.when(nxt >= NB_S)
                def _():
                    scat_desc(nxt - NB_S, bn).wait()

                load_desc(nxt, bn).start()

            load_desc(i, b).wait()
            scat_desc(i, b).start(add=True)
            return carry

        lax.fori_loop(0, SCATTER_ITERS, body, 0)
        for j in range(NB_S):
            b = (SCATTER_ITERS - NB_S + j) % NB_S
            scat_desc(SCATTER_ITERS - NB_S + j, b).wait()
        plsc.subcore_barrier()

        for kk in range(ROWS_PER_TILE // K):
            r0 = pl.multiple_of(my_rows + kk * K, K)
            pltpu.sync_copy(acc_sh.at[pl.ds(r0, K)], buf_v.at[0])
            pltpu.sync_copy(buf_v.at[0], agg_hbm.at[c, pl.ds(r0, K), ch])
        plsc.subcore_barrier()


def _mlp_body(g_ref, ea_ref, dir_ref, wx_ref, we_ref, b_ref,
              o0_ref, o1_ref):
    acc = jnp.dot(g_ref[...].astype(jnp.bfloat16), wx_ref[...],
                  preferred_element_type=jnp.float32)
    acc = acc + jnp.dot(ea_ref[...].astype(jnp.bfloat16), we_ref[...],
                        preferred_element_type=jnp.float32)
    acc = jnp.maximum(acc + b_ref[...], 0.0)
    hsel = jnp.where(dir_ref[...] > 0, acc[:, :D_FEAT], acc[:, D_FEAT:])
    o0_ref[...] = hsel[:, :K]
    o1_ref[...] = hsel[:, K:]


_H_TYPE = jax.ShapeDtypeStruct((E_PAD, K), jnp.float32)


def _edge_mlp(g, ea, dirf, wx, we, b):
    be = 512
    grid = (E_PAD // be,)
    return pl.pallas_call(
        _mlp_body,
        grid=grid,
        in_specs=[
            pl.BlockSpec((be, D_FEAT), lambda i: (i, 0)),
            pl.BlockSpec((be, D_EDGE), lambda i: (i, 0)),
            pl.BlockSpec((be, 1), lambda i: (i, 0)),
            pl.BlockSpec((D_FEAT, 2 * D_FEAT), lambda i: (0, 0)),
            pl.BlockSpec((D_EDGE, 2 * D_FEAT), lambda i: (0, 0)),
            pl.BlockSpec((1, 2 * D_FEAT), lambda i: (0, 0)),
        ],
        out_specs=[pl.BlockSpec((be, K), lambda i: (i, 0))] * 2,
        out_shape=[_H_TYPE] * 2,
    )(g, ea, dirf, wx, we, b)


def _node_body(ain_ref, aout_ref, wt_ref, wb_ref, b_ref, o_ref):
    acc = jnp.dot(ain_ref[...], wt_ref[...],
                  preferred_element_type=jnp.float32)
    acc = acc + jnp.dot(aout_ref[...], wb_ref[...],
                        preferred_element_type=jnp.float32)
    o_ref[...] = jnp.maximum(acc + b_ref[...], 0.0)


def _node_mlp(agg, wt, wb, b):
    bn = 512
    nblk = N_PAD // bn
    grid = (nblk,)
    return pl.pallas_call(
        _node_body,
        grid=grid,
        in_specs=[
            pl.BlockSpec((bn, 2 * K), lambda i: (i, 0)),
            pl.BlockSpec((bn, 2 * K), lambda i: (i + nblk, 0)),
            pl.BlockSpec((D_FEAT, D_FEAT), lambda i: (0, 0)),
            pl.BlockSpec((D_FEAT, D_FEAT), lambda i: (0, 0)),
            pl.BlockSpec((1, D_FEAT), lambda i: (0, 0)),
        ],
        out_specs=pl.BlockSpec((bn, D_FEAT), lambda i: (i, 0)),
        out_shape=jax.ShapeDtypeStruct((N_PAD, D_FEAT), jnp.float32),
    )(agg, agg, wt, wb, b)


def kernel(x, edge_index, edge_attr, W_out, b_out, W_in, b_in, W_node, b_node):
    row = edge_index[0]
    col = edge_index[1]
    n_e = row.shape[0]
    pad_e = E_PAD - n_e

    spread = N_NODES + (jnp.arange(n_e, dtype=jnp.int32) & 127)
    t_in = jnp.where(row > col, row, spread)
    t_out = jnp.where(row < col, row, spread)
    dump = N_NODES + (jnp.arange(pad_e, dtype=jnp.int32) & 127)
    tsel = jnp.stack([jnp.concatenate([t_in, dump]),
                      jnp.concatenate([t_out, dump])])
    tsel4 = tsel.reshape(NC, NS, SCATTER_ITERS, K)
    dirf = jnp.concatenate([(row > col).astype(jnp.float32),
                            jnp.zeros((pad_e,), jnp.float32)])
    dirf = dirf.reshape(E_PAD, 1)
    col_p = jnp.concatenate([col, jnp.zeros((pad_e,), jnp.int32)])
    ea_p = jnp.concatenate(
        [edge_attr, jnp.zeros((pad_e, D_EDGE), jnp.float32)])

    wcat = jnp.concatenate([W_in, W_out], axis=1)
    wx = wcat[:D_FEAT].astype(jnp.bfloat16)
    we = wcat[D_FEAT:].astype(jnp.bfloat16)
    bcat = jnp.concatenate([b_in, b_out]).reshape(1, 2 * D_FEAT)

    g = _sc_gather(x, col_p)
    hA, hB = _edge_mlp(g, ea_p, dirf, wx, we, bcat)
    agg = _sc_scatter(hA, hB, tsel4)
    agg2 = agg.reshape(NC * N_PAD, 2 * K)
    out = _node_mlp(agg2, W_node[:D_FEAT], W_node[D_FEAT:],
                    b_node.reshape(1, D_FEAT))
    return out[:N_NODES]


# speedup vs baseline: 1.0410x; 1.0410x over previous
import functools

import jax
import jax.numpy as jnp
from jax import lax
from jax.experimental import pallas as pl
from jax.experimental.pallas import tpu as pltpu
from jax.experimental.pallas import tpu_sc as plsc

N_NODES = 10000
D_FEAT = 256
D_EDGE = 16

NC, NS, LANES = 2, 16, 16
NW = NC * NS
K = 128
E_PAD = 163840
GATHER_ITERS = E_PAD // (NW * K)
E_PER_TILE = E_PAD // NS
SCATTER_ITERS = E_PER_TILE // K
N_PAD = 10240

_MESH = plsc.VectorSubcoreMesh(core_axis_name="c", subcore_axis_name="s")
NB = 3
NB_S = 2
CHUNK = GATHER_ITERS * K


PAIR_ITERS = 2 * GATHER_ITERS
IT0 = 55
ITMAX = max(IT0, PAIR_ITERS - IT0)
PAIR_E = PAIR_ITERS * K


@functools.partial(
    pl.kernel,
    mesh=_MESH,
    out_type=jax.ShapeDtypeStruct((E_PAD, D_FEAT), jnp.float32),
    scratch_types=[
        pltpu.VMEM((PAIR_E,), jnp.int32),
        pltpu.VMEM((NB, K, D_FEAT), jnp.float32),
        pltpu.SemaphoreType.DMA((NB,)),
        pltpu.SemaphoreType.DMA((NB,)),
    ],
)
def _sc_gather(x_hbm, col_hbm, g_hbm, idx_v, rows_v, sem_g, sem_s):
    c = lax.axis_index("c")
    s = lax.axis_index("s")
    pair0 = pl.multiple_of(s * PAIR_E, K)
    ib = pl.multiple_of(c * (IT0 * K), K)
    base0 = pl.multiple_of(pair0 + ib, K)
    iters = jnp.where(c == 0, IT0, PAIR_ITERS - IT0)
    pltpu.sync_copy(col_hbm.at[pl.ds(pair0, PAIR_E)], idx_v)

    def gather_desc(i, b):
        off = pl.multiple_of(ib + i * K, K)
        return pltpu.make_async_copy(
            x_hbm.at[idx_v.at[pl.ds(off, K)]], rows_v.at[b], sem_g.at[b])

    def store_desc(i, b):
        off = pl.multiple_of(base0 + i * K, K)
        return pltpu.make_async_copy(
            rows_v.at[b], g_hbm.at[pl.ds(off, K)], sem_s.at[b])

    gather_desc(0, 0).start()

    def body(i, carry):
        b = lax.rem(i, NB)
        nxt = i + 1

        @pl.when(nxt < iters)
        def _():
            bn = lax.rem(nxt, NB)

            @pl.when(nxt >= NB)
            def _():
                store_desc(nxt - NB, bn).wait()

            gather_desc(nxt, bn).start()

        @pl.when(i < iters)
        def _():
            gather_desc(i, b).wait()
            store_desc(i, b).start()
        return carry

    lax.fori_loop(0, ITMAX, body, 0)
    for j in range(NB):
        i = iters - NB + j
        store_desc(i, lax.rem(i, NB)).wait()


ROWS_PER_TILE = N_PAD // NS


@functools.partial(
    pl.kernel,
    mesh=_MESH,
    out_type=jax.ShapeDtypeStruct((NC, N_PAD, 2, K), jnp.float32),
    scratch_types=[
        pltpu.VMEM_SHARED((N_PAD, K), jnp.float32),
        pltpu.VMEM((SCATTER_ITERS, K), jnp.int32),
        pltpu.VMEM((NB_S, K, K), jnp.float32),
        pltpu.SemaphoreType.DMA((NB_S,)),
        pltpu.SemaphoreType.DMA((NB_S,)),
    ],
)
def _sc_scatter(hA, hB, tsel_hbm, agg_hbm,
                acc_sh, idx_v, buf_v, sem_l, sem_sc):
    c = lax.axis_index("c")
    s = lax.axis_index("s")
    my_rows = pl.multiple_of(s * ROWS_PER_TILE, K)
    ebase0 = pl.multiple_of(s * E_PER_TILE, K)
    zeros16 = jnp.zeros((LANES,), jnp.float32)

    pltpu.sync_copy(tsel_hbm.at[c, s], idx_v)

    for ch, h_hbm in ((0, hA), (1, hB)):
        def load_desc(i, b, h_hbm=h_hbm):
            off = pl.multiple_of(ebase0 + i * K, K)
            return pltpu.make_async_copy(
                h_hbm.at[pl.ds(off, K)], buf_v.at[b], sem_l.at[b])

        def scat_desc(i, b):
            return pltpu.make_async_copy(
                buf_v.at[b], acc_sh.at[idx_v.at[i]], sem_sc.at[b])

        def zbody(r, carry):
            for j in range(K // LANES):
                buf_v[0, r, pl.ds(j * LANES, LANES)] = zeros16
            return carry

        lax.fori_loop(0, K, zbody, 0)
        for kk in range(ROWS_PER_TILE // K):
            pltpu.sync_copy(buf_v.at[0],
                            acc_sh.at[pl.ds(my_rows + kk * K, K)])
        plsc.subcore_barrier()

        load_desc(0, 0).start()

        def body(i, carry):
            b = lax.rem(i, NB_S)
            nxt = i + 1

            @pl.when(nxt < SCATTER_ITERS)
            def _():
                bn = lax.rem(nxt, NB_S)

                @pl.when(nxt >= NB_S)
                def _():
                    scat_desc(nxt - NB_S, bn).wait()

                load_desc(nxt, bn).start()

            load_desc(i, b).wait()
            scat_desc(i, b).start(add=True)
            return carry

        lax.fori_loop(0, SCATTER_ITERS, body, 0)
        for j in range(NB_S):
            b = (SCATTER_ITERS - NB_S + j) % NB_S
            scat_desc(SCATTER_ITERS - NB_S + j, b).wait()
        plsc.subcore_barrier()

        for kk in range(ROWS_PER_TILE // K):
            r0 = pl.multiple_of(my_rows + kk * K, K)
            pltpu.sync_copy(acc_sh.at[pl.ds(r0, K)], buf_v.at[0])
            pltpu.sync_copy(buf_v.at[0], agg_hbm.at[c, pl.ds(r0, K), ch])
        plsc.subcore_barrier()


def _mlp_body(g_ref, ea_ref, dir_ref, wx_ref, we_ref, b_ref,
              o0_ref, o1_ref):
    acc = jnp.dot(g_ref[...].astype(jnp.bfloat16), wx_ref[...],
                  preferred_element_type=jnp.float32)
    acc = acc + jnp.dot(ea_ref[...].astype(jnp.bfloat16), we_ref[...],
                        preferred_element_type=jnp.float32)
    acc = jnp.maximum(acc + b_ref[...], 0.0)
    hsel = jnp.where(dir_ref[...] > 0, acc[:, :D_FEAT], acc[:, D_FEAT:])
    o0_ref[...] = hsel[:, :K]
    o1_ref[...] = hsel[:, K:]


_H_TYPE = jax.ShapeDtypeStruct((E_PAD, K), jnp.float32)


def _edge_mlp(g, ea, dirf, wx, we, b):
    be = 512
    grid = (E_PAD // be,)
    return pl.pallas_call(
        _mlp_body,
        grid=grid,
        in_specs=[
            pl.BlockSpec((be, D_FEAT), lambda i: (i, 0)),
            pl.BlockSpec((be, D_EDGE), lambda i: (i, 0)),
            pl.BlockSpec((be, 1), lambda i: (i, 0)),
            pl.BlockSpec((D_FEAT, 2 * D_FEAT), lambda i: (0, 0)),
            pl.BlockSpec((D_EDGE, 2 * D_FEAT), lambda i: (0, 0)),
            pl.BlockSpec((1, 2 * D_FEAT), lambda i: (0, 0)),
        ],
        out_specs=[pl.BlockSpec((be, K), lambda i: (i, 0))] * 2,
        out_shape=[_H_TYPE] * 2,
    )(g, ea, dirf, wx, we, b)


def _node_body(ain_ref, aout_ref, wt_ref, wb_ref, b_ref, o_ref):
    acc = jnp.dot(ain_ref[...], wt_ref[...],
                  preferred_element_type=jnp.float32)
    acc = acc + jnp.dot(aout_ref[...], wb_ref[...],
                        preferred_element_type=jnp.float32)
    o_ref[...] = jnp.maximum(acc + b_ref[...], 0.0)


def _node_mlp(agg, wt, wb, b):
    bn = 512
    nblk = N_PAD // bn
    grid = (nblk,)
    return pl.pallas_call(
        _node_body,
        grid=grid,
        in_specs=[
            pl.BlockSpec((bn, 2 * K), lambda i: (i, 0)),
            pl.BlockSpec((bn, 2 * K), lambda i: (i + nblk, 0)),
            pl.BlockSpec((D_FEAT, D_FEAT), lambda i: (0, 0)),
            pl.BlockSpec((D_FEAT, D_FEAT), lambda i: (0, 0)),
            pl.BlockSpec((1, D_FEAT), lambda i: (0, 0)),
        ],
        out_specs=pl.BlockSpec((bn, D_FEAT), lambda i: (i, 0)),
        out_shape=jax.ShapeDtypeStruct((N_PAD, D_FEAT), jnp.float32),
    )(agg, agg, wt, wb, b)


def kernel(x, edge_index, edge_attr, W_out, b_out, W_in, b_in, W_node, b_node):
    row = edge_index[0]
    col = edge_index[1]
    n_e = row.shape[0]
    pad_e = E_PAD - n_e

    spread = N_NODES + (jnp.arange(n_e, dtype=jnp.int32) & 127)
    t_in = jnp.where(row > col, row, spread)
    t_out = jnp.where(row < col, row, spread)
    dump = N_NODES + (jnp.arange(pad_e, dtype=jnp.int32) & 127)
    tsel = jnp.stack([jnp.concatenate([t_in, dump]),
                      jnp.concatenate([t_out, dump])])
    tsel4 = tsel.reshape(NC, NS, SCATTER_ITERS, K)
    dirf = jnp.concatenate([(row > col).astype(jnp.float32),
                            jnp.zeros((pad_e,), jnp.float32)])
    dirf = dirf.reshape(E_PAD, 1)
    col_p = jnp.concatenate([col, jnp.zeros((pad_e,), jnp.int32)])
    ea_p = jnp.concatenate(
        [edge_attr, jnp.zeros((pad_e, D_EDGE), jnp.float32)])

    wcat = jnp.concatenate([W_in, W_out], axis=1)
    wx = wcat[:D_FEAT].astype(jnp.bfloat16)
    we = wcat[D_FEAT:].astype(jnp.bfloat16)
    bcat = jnp.concatenate([b_in, b_out]).reshape(1, 2 * D_FEAT)

    g = _sc_gather(x, col_p)
    hA, hB = _edge_mlp(g, ea_p, dirf, wx, we, bcat)
    agg = _sc_scatter(hA, hB, tsel4)
    agg2 = agg.reshape(NC * N_PAD, 2 * K)
    out = _node_mlp(agg2, W_node[:D_FEAT], W_node[D_FEAT:],
                    b_node.reshape(1, D_FEAT))
    return out[:N_NODES]


# speedup vs baseline: 1.1477x; 1.1024x over previous
import functools

import jax
import jax.numpy as jnp
from jax import lax
from jax.experimental import pallas as pl
from jax.experimental.pallas import tpu as pltpu
from jax.experimental.pallas import tpu_sc as plsc

N_NODES = 10000
D_FEAT = 256
D_EDGE = 16

NC, NS, LANES = 2, 16, 16
NW = NC * NS
K = 128
E_PAD = 163840
GATHER_ITERS = E_PAD // (NW * K)
E_PER_TILE = E_PAD // NS
SCATTER_ITERS = E_PER_TILE // K
N_PAD = 10240

_MESH = plsc.VectorSubcoreMesh(core_axis_name="c", subcore_axis_name="s")
NB = 3
NB_S = 2
CHUNK = GATHER_ITERS * K


IT0_FRAC = 55 / 80


def _make_gather(e_size):
    pair_iters = e_size // (NS * K)
    it0 = int(round(pair_iters * IT0_FRAC))
    it0 = min(max(it0, NB), pair_iters - NB)
    itmax = max(it0, pair_iters - it0)
    pair_e = pair_iters * K

    @functools.partial(
        pl.kernel,
        mesh=_MESH,
        out_type=jax.ShapeDtypeStruct((e_size, D_FEAT), jnp.float32),
        scratch_types=[
            pltpu.VMEM((pair_e,), jnp.int32),
            pltpu.VMEM((NB, K, D_FEAT), jnp.float32),
            pltpu.SemaphoreType.DMA((NB,)),
            pltpu.SemaphoreType.DMA((NB,)),
        ],
    )
    def _sc_gather(x_hbm, col_hbm, g_hbm, idx_v, rows_v, sem_g, sem_s):
        c = lax.axis_index("c")
        s = lax.axis_index("s")
        pair0 = pl.multiple_of(s * pair_e, K)
        ib = pl.multiple_of(c * (it0 * K), K)
        base0 = pl.multiple_of(pair0 + ib, K)
        iters = jnp.where(c == 0, it0, pair_iters - it0)
        pltpu.sync_copy(col_hbm.at[pl.ds(pair0, pair_e)], idx_v)

        def gather_desc(i, b):
            off = pl.multiple_of(ib + i * K, K)
            return pltpu.make_async_copy(
                x_hbm.at[idx_v.at[pl.ds(off, K)]], rows_v.at[b], sem_g.at[b])

        def store_desc(i, b):
            off = pl.multiple_of(base0 + i * K, K)
            return pltpu.make_async_copy(
                rows_v.at[b], g_hbm.at[pl.ds(off, K)], sem_s.at[b])

        gather_desc(0, 0).start()

        def body(i, carry):
            b = lax.rem(i, NB)
            nxt = i + 1

            @pl.when(nxt < iters)
            def _():
                bn = lax.rem(nxt, NB)

                @pl.when(nxt >= NB)
                def _():
                    store_desc(nxt - NB, bn).wait()

                gather_desc(nxt, bn).start()

            @pl.when(i < iters)
            def _():
                gather_desc(i, b).wait()
                store_desc(i, b).start()
            return carry

        lax.fori_loop(0, itmax, body, 0)
        for j in range(NB):
            i = iters - NB + j
            store_desc(i, lax.rem(i, NB)).wait()

    return _sc_gather


ROWS_PER_TILE = N_PAD // NS


def _make_scatter(e_size):
    e_per_tile = e_size // NS
    sc_iters = e_per_tile // K

    @functools.partial(
        pl.kernel,
        mesh=_MESH,
        out_type=jax.ShapeDtypeStruct((NC, N_PAD, 2, K), jnp.float32),
        scratch_types=[
            pltpu.VMEM_SHARED((N_PAD, K), jnp.float32),
            pltpu.VMEM((sc_iters, K), jnp.int32),
            pltpu.VMEM((NB_S, K, K), jnp.float32),
            pltpu.SemaphoreType.DMA((NB_S,)),
            pltpu.SemaphoreType.DMA((NB_S,)),
        ],
    )
    def _sc_scatter(hA, hB, tsel_hbm, agg_hbm,
                    acc_sh, idx_v, buf_v, sem_l, sem_sc):
        c = lax.axis_index("c")
        s = lax.axis_index("s")
        my_rows = pl.multiple_of(s * ROWS_PER_TILE, K)
        ebase0 = pl.multiple_of(s * e_per_tile, K)
        zeros16 = jnp.zeros((LANES,), jnp.float32)

        pltpu.sync_copy(tsel_hbm.at[c, s], idx_v)

        for ch, h_hbm in ((0, hA), (1, hB)):
            def load_desc(i, b, h_hbm=h_hbm):
                off = pl.multiple_of(ebase0 + i * K, K)
                return pltpu.make_async_copy(
                    h_hbm.at[pl.ds(off, K)], buf_v.at[b], sem_l.at[b])

            def scat_desc(i, b):
                return pltpu.make_async_copy(
                    buf_v.at[b], acc_sh.at[idx_v.at[i]], sem_sc.at[b])

            def zbody(r, carry):
                for j in range(K // LANES):
                    buf_v[0, r, pl.ds(j * LANES, LANES)] = zeros16
                return carry

            lax.fori_loop(0, K, zbody, 0)
            for kk in range(ROWS_PER_TILE // K):
                pltpu.sync_copy(buf_v.at[0],
                                acc_sh.at[pl.ds(my_rows + kk * K, K)])
            plsc.subcore_barrier()

            load_desc(0, 0).start()

            def body(i, carry):
                b = lax.rem(i, NB_S)
                nxt = i + 1

                @pl.when(nxt < sc_iters)
                def _():
                    bn = lax.rem(nxt, NB_S)

                    @pl.when(nxt >= NB_S)
                    def _():
                        scat_desc(nxt - NB_S, bn).wait()

                    load_desc(nxt, bn).start()

                load_desc(i, b).wait()
                scat_desc(i, b).start(add=True)
                return carry

            lax.fori_loop(0, sc_iters, body, 0)
            for j in range(NB_S):
                b = (sc_iters - NB_S + j) % NB_S
                scat_desc(sc_iters - NB_S + j, b).wait()
            plsc.subcore_barrier()

            for kk in range(ROWS_PER_TILE // K):
                r0 = pl.multiple_of(my_rows + kk * K, K)
                pltpu.sync_copy(acc_sh.at[pl.ds(r0, K)], buf_v.at[0])
                pltpu.sync_copy(buf_v.at[0], agg_hbm.at[c, pl.ds(r0, K), ch])
            plsc.subcore_barrier()

    return _sc_scatter


def _mlp_body(g_ref, ea_ref, dir_ref, wx_ref, we_ref, b_ref,
              o0_ref, o1_ref):
    acc = jnp.dot(g_ref[...].astype(jnp.bfloat16), wx_ref[...],
                  preferred_element_type=jnp.float32)
    acc = acc + jnp.dot(ea_ref[...].astype(jnp.bfloat16), we_ref[...],
                        preferred_element_type=jnp.float32)
    acc = jnp.maximum(acc + b_ref[...], 0.0)
    hsel = jnp.where(dir_ref[...] > 0, acc[:, :D_FEAT], acc[:, D_FEAT:])
    o0_ref[...] = hsel[:, :K]
    o1_ref[...] = hsel[:, K:]


def _edge_mlp(g, ea, dirf, wx, we, b):
    e_size = g.shape[0]
    be = 512
    grid = (e_size // be,)
    return pl.pallas_call(
        _mlp_body,
        grid=grid,
        in_specs=[
            pl.BlockSpec((be, D_FEAT), lambda i: (i, 0)),
            pl.BlockSpec((be, D_EDGE), lambda i: (i, 0)),
            pl.BlockSpec((be, 1), lambda i: (i, 0)),
            pl.BlockSpec((D_FEAT, 2 * D_FEAT), lambda i: (0, 0)),
            pl.BlockSpec((D_EDGE, 2 * D_FEAT), lambda i: (0, 0)),
            pl.BlockSpec((1, 2 * D_FEAT), lambda i: (0, 0)),
        ],
        out_specs=[pl.BlockSpec((be, K), lambda i: (i, 0))] * 2,
        out_shape=[jax.ShapeDtypeStruct((e_size, K), jnp.float32)] * 2,
    )(g, ea, dirf, wx, we, b)


def _node_body(ainA, aoutA, ainB, aoutB, wt_ref, wb_ref, b_ref, o_ref):
    acc = jnp.dot(ainA[...] + ainB[...], wt_ref[...],
                  preferred_element_type=jnp.float32)
    acc = acc + jnp.dot(aoutA[...] + aoutB[...], wb_ref[...],
                        preferred_element_type=jnp.float32)
    o_ref[...] = jnp.maximum(acc + b_ref[...], 0.0)


def _node_mlp(aggA, aggB, wt, wb, b):
    bn = 512
    nblk = N_PAD // bn
    grid = (nblk,)
    sec = [
        pl.BlockSpec((bn, 2 * K), lambda i: (i, 0)),
        pl.BlockSpec((bn, 2 * K), lambda i: (i + nblk, 0)),
    ]
    return pl.pallas_call(
        _node_body,
        grid=grid,
        in_specs=sec + sec + [
            pl.BlockSpec((D_FEAT, D_FEAT), lambda i: (0, 0)),
            pl.BlockSpec((D_FEAT, D_FEAT), lambda i: (0, 0)),
            pl.BlockSpec((1, D_FEAT), lambda i: (0, 0)),
        ],
        out_specs=pl.BlockSpec((bn, D_FEAT), lambda i: (i, 0)),
        out_shape=jax.ShapeDtypeStruct((N_PAD, D_FEAT), jnp.float32),
    )(aggA, aggA, aggB, aggB, wt, wb, b)


E_H = E_PAD // 2
_gather_h = _make_gather(E_H)
_scatter_h = _make_scatter(E_H)


def kernel(x, edge_index, edge_attr, W_out, b_out, W_in, b_in, W_node, b_node):
    row = edge_index[0]
    col = edge_index[1]
    n_e = row.shape[0]
    pad_e = E_PAD - n_e

    spread = N_NODES + (jnp.arange(n_e, dtype=jnp.int32) & 127)
    t_in = jnp.where(row > col, row, spread)
    t_out = jnp.where(row < col, row, spread)
    dump = N_NODES + (jnp.arange(pad_e, dtype=jnp.int32) & 127)
    tsel = jnp.stack([jnp.concatenate([t_in, dump]),
                      jnp.concatenate([t_out, dump])])
    dirf = jnp.concatenate([(row > col).astype(jnp.float32),
                            jnp.zeros((pad_e,), jnp.float32)])
    dirf = dirf.reshape(E_PAD, 1)
    col_p = jnp.concatenate([col, jnp.zeros((pad_e,), jnp.int32)])
    ea_p = jnp.concatenate(
        [edge_attr, jnp.zeros((pad_e, D_EDGE), jnp.float32)])

    wcat = jnp.concatenate([W_in, W_out], axis=1)
    wx = wcat[:D_FEAT].astype(jnp.bfloat16)
    we = wcat[D_FEAT:].astype(jnp.bfloat16)
    bcat = jnp.concatenate([b_in, b_out]).reshape(1, 2 * D_FEAT)

    sc_it_h = E_H // NS // K
    aggs = []
    hs = []
    for ph in range(2):
        e0 = ph * E_H
        g = _gather_h(x, lax.dynamic_slice_in_dim(col_p, e0, E_H))
        hs.append(_edge_mlp(
            g, lax.dynamic_slice_in_dim(ea_p, e0, E_H),
            lax.dynamic_slice_in_dim(dirf, e0, E_H), wx, we, bcat))
    for ph in range(2):
        tsel4 = tsel[:, ph * E_H:(ph + 1) * E_H].reshape(NC, NS, sc_it_h, K)
        hA, hB = hs[ph]
        agg = _scatter_h(hA, hB, tsel4)
        aggs.append(agg.reshape(NC * N_PAD, 2 * K))
    out = _node_mlp(aggs[0], aggs[1], W_node[:D_FEAT], W_node[D_FEAT:],
                    b_node.reshape(1, D_FEAT))
    return out[:N_NODES]


# speedup vs baseline: 1.2498x; 1.0890x over previous
import functools

import jax
import jax.numpy as jnp
from jax import lax
from jax.experimental import pallas as pl
from jax.experimental.pallas import tpu as pltpu
from jax.experimental.pallas import tpu_sc as plsc

N_NODES = 10000
D_FEAT = 256
D_EDGE = 16

NC, NS, LANES = 2, 16, 16
NW = NC * NS
K = 128
E_PAD = 163840
GATHER_ITERS = E_PAD // (NW * K)
E_PER_TILE = E_PAD // NS
SCATTER_ITERS = E_PER_TILE // K
N_PAD = 10240

_MESH = plsc.VectorSubcoreMesh(core_axis_name="c", subcore_axis_name="s")
NB = 3
NB_S = 2
CHUNK = GATHER_ITERS * K


IT0_FRAC = 55 / 80


def _make_gather(e_size):
    pair_iters = e_size // (NS * K)
    it0 = int(round(pair_iters * IT0_FRAC))
    it0 = min(max(it0, NB), pair_iters - NB)
    itmax = max(it0, pair_iters - it0)
    pair_e = pair_iters * K

    @functools.partial(
        pl.kernel,
        mesh=_MESH,
        out_type=jax.ShapeDtypeStruct((e_size, D_FEAT), jnp.float32),
        scratch_types=[
            pltpu.VMEM((pair_e,), jnp.int32),
            pltpu.VMEM((NB, K, D_FEAT), jnp.float32),
            pltpu.SemaphoreType.DMA((NB,)),
            pltpu.SemaphoreType.DMA((NB,)),
        ],
    )
    def _sc_gather(x_hbm, col_hbm, g_hbm, idx_v, rows_v, sem_g, sem_s):
        c = lax.axis_index("c")
        s = lax.axis_index("s")
        pair0 = pl.multiple_of(s * pair_e, K)
        ib = pl.multiple_of(c * (it0 * K), K)
        base0 = pl.multiple_of(pair0 + ib, K)
        iters = jnp.where(c == 0, it0, pair_iters - it0)
        pltpu.sync_copy(col_hbm.at[pl.ds(pair0, pair_e)], idx_v)

        def gather_desc(i, b):
            off = pl.multiple_of(ib + i * K, K)
            return pltpu.make_async_copy(
                x_hbm.at[idx_v.at[pl.ds(off, K)]], rows_v.at[b], sem_g.at[b])

        def store_desc(i, b):
            off = pl.multiple_of(base0 + i * K, K)
            return pltpu.make_async_copy(
                rows_v.at[b], g_hbm.at[pl.ds(off, K)], sem_s.at[b])

        gather_desc(0, 0).start()

        def body(i, carry):
            b = lax.rem(i, NB)
            nxt = i + 1

            @pl.when(nxt < iters)
            def _():
                bn = lax.rem(nxt, NB)

                @pl.when(nxt >= NB)
                def _():
                    store_desc(nxt - NB, bn).wait()

                gather_desc(nxt, bn).start()

            @pl.when(i < iters)
            def _():
                gather_desc(i, b).wait()
                store_desc(i, b).start()
            return carry

        lax.fori_loop(0, itmax, body, 0)
        for j in range(NB):
            i = iters - NB + j
            store_desc(i, lax.rem(i, NB)).wait()

    return _sc_gather


ROWS_PER_TILE = N_PAD // NS


def _make_scatter(e_size):
    e_per_tile = e_size // NS
    sc_iters = e_per_tile // K

    @functools.partial(
        pl.kernel,
        mesh=_MESH,
        out_type=jax.ShapeDtypeStruct((NC, N_PAD, 2, K), jnp.float32),
        scratch_types=[
            pltpu.VMEM_SHARED((N_PAD, K), jnp.float32),
            pltpu.VMEM((sc_iters, K), jnp.int32),
            pltpu.VMEM((NB_S, K, K), jnp.float32),
            pltpu.SemaphoreType.DMA((NB_S,)),
            pltpu.SemaphoreType.DMA((NB_S,)),
        ],
    )
    def _sc_scatter(hA, hB, tsel_hbm, agg_hbm,
                    acc_sh, idx_v, buf_v, sem_l, sem_sc):
        c = lax.axis_index("c")
        s = lax.axis_index("s")
        my_rows = pl.multiple_of(s * ROWS_PER_TILE, K)
        ebase0 = pl.multiple_of(s * e_per_tile, K)
        zeros16 = jnp.zeros((LANES,), jnp.float32)

        pltpu.sync_copy(tsel_hbm.at[c, s], idx_v)

        for ch, h_hbm in ((0, hA), (1, hB)):
            def load_desc(i, b, h_hbm=h_hbm):
                off = pl.multiple_of(ebase0 + i * K, K)
                return pltpu.make_async_copy(
                    h_hbm.at[pl.ds(off, K)], buf_v.at[b], sem_l.at[b])

            def scat_desc(i, b):
                return pltpu.make_async_copy(
                    buf_v.at[b], acc_sh.at[idx_v.at[i]], sem_sc.at[b])

            def zbody(r, carry):
                for j in range(K // LANES):
                    buf_v[0, r, pl.ds(j * LANES, LANES)] = zeros16
                return carry

            lax.fori_loop(0, K, zbody, 0)
            for kk in range(ROWS_PER_TILE // K):
                pltpu.sync_copy(buf_v.at[0],
                                acc_sh.at[pl.ds(my_rows + kk * K, K)])
            plsc.subcore_barrier()

            load_desc(0, 0).start()

            def body(i, carry):
                b = lax.rem(i, NB_S)
                nxt = i + 1

                @pl.when(nxt < sc_iters)
                def _():
                    bn = lax.rem(nxt, NB_S)

                    @pl.when(nxt >= NB_S)
                    def _():
                        scat_desc(nxt - NB_S, bn).wait()

                    load_desc(nxt, bn).start()

                load_desc(i, b).wait()
                scat_desc(i, b).start(add=True)
                return carry

            lax.fori_loop(0, sc_iters, body, 0)
            for j in range(NB_S):
                b = (sc_iters - NB_S + j) % NB_S
                scat_desc(sc_iters - NB_S + j, b).wait()
            plsc.subcore_barrier()

            for kk in range(ROWS_PER_TILE // K):
                r0 = pl.multiple_of(my_rows + kk * K, K)
                pltpu.sync_copy(acc_sh.at[pl.ds(r0, K)], buf_v.at[0])
                pltpu.sync_copy(buf_v.at[0], agg_hbm.at[c, pl.ds(r0, K), ch])
            plsc.subcore_barrier()

    return _sc_scatter


def _mlp_body(g_ref, ea_ref, dir_ref, wx_ref, we_ref, b_ref,
              o0_ref, o1_ref):
    acc = jnp.dot(g_ref[...].astype(jnp.bfloat16), wx_ref[...],
                  preferred_element_type=jnp.float32)
    acc = acc + jnp.dot(ea_ref[...].astype(jnp.bfloat16), we_ref[...],
                        preferred_element_type=jnp.float32)
    acc = jnp.maximum(acc + b_ref[...], 0.0)
    hsel = jnp.where(dir_ref[...] > 0, acc[:, :D_FEAT], acc[:, D_FEAT:])
    o0_ref[...] = hsel[:, :K]
    o1_ref[...] = hsel[:, K:]


def _edge_mlp(g, ea, dirf, wx, we, b):
    e_size = g.shape[0]
    be = 1024
    grid = (e_size // be,)
    return pl.pallas_call(
        _mlp_body,
        grid=grid,
        in_specs=[
            pl.BlockSpec((be, D_FEAT), lambda i: (i, 0)),
            pl.BlockSpec((be, D_EDGE), lambda i: (i, 0)),
            pl.BlockSpec((be, 1), lambda i: (i, 0)),
            pl.BlockSpec((D_FEAT, 2 * D_FEAT), lambda i: (0, 0)),
            pl.BlockSpec((D_EDGE, 2 * D_FEAT), lambda i: (0, 0)),
            pl.BlockSpec((1, 2 * D_FEAT), lambda i: (0, 0)),
        ],
        out_specs=[pl.BlockSpec((be, K), lambda i: (i, 0))] * 2,
        out_shape=[jax.ShapeDtypeStruct((e_size, K), jnp.float32)] * 2,
    )(g, ea, dirf, wx, we, b)


def _node_body(ainA, aoutA, ainB, aoutB, wt_ref, wb_ref, b_ref, o_ref):
    acc = jnp.dot(ainA[...] + ainB[...], wt_ref[...],
                  preferred_element_type=jnp.float32)
    acc = acc + jnp.dot(aoutA[...] + aoutB[...], wb_ref[...],
                        preferred_element_type=jnp.float32)
    o_ref[...] = jnp.maximum(acc + b_ref[...], 0.0)


def _node_mlp(aggA, aggB, wt, wb, b):
    bn = 512
    nblk = N_PAD // bn
    grid = (nblk,)
    sec = [
        pl.BlockSpec((bn, 2 * K), lambda i: (i, 0)),
        pl.BlockSpec((bn, 2 * K), lambda i: (i + nblk, 0)),
    ]
    return pl.pallas_call(
        _node_body,
        grid=grid,
        in_specs=sec + sec + [
            pl.BlockSpec((D_FEAT, D_FEAT), lambda i: (0, 0)),
            pl.BlockSpec((D_FEAT, D_FEAT), lambda i: (0, 0)),
            pl.BlockSpec((1, D_FEAT), lambda i: (0, 0)),
        ],
        out_specs=pl.BlockSpec((bn, D_FEAT), lambda i: (i, 0)),
        out_shape=jax.ShapeDtypeStruct((N_PAD, D_FEAT), jnp.float32),
    )(aggA, aggA, aggB, aggB, wt, wb, b)


E_H = E_PAD // 2
_gather_h = _make_gather(E_H)
_scatter_h = _make_scatter(E_H)


def kernel(x, edge_index, edge_attr, W_out, b_out, W_in, b_in, W_node, b_node):
    row = edge_index[0]
    col = edge_index[1]
    n_e = row.shape[0]
    pad_e = E_PAD - n_e

    spread = N_NODES + (jnp.arange(n_e, dtype=jnp.int32) & 127)
    t_in = jnp.where(row > col, row, spread)
    t_out = jnp.where(row < col, row, spread)
    dump = N_NODES + (jnp.arange(pad_e, dtype=jnp.int32) & 127)
    tsel = jnp.stack([jnp.concatenate([t_in, dump]),
                      jnp.concatenate([t_out, dump])])
    dirf = jnp.concatenate([(row > col).astype(jnp.float32),
                            jnp.zeros((pad_e,), jnp.float32)])
    dirf = dirf.reshape(E_PAD, 1)
    col_p = jnp.concatenate([col, jnp.zeros((pad_e,), jnp.int32)])
    ea_p = jnp.concatenate(
        [edge_attr, jnp.zeros((pad_e, D_EDGE), jnp.float32)])

    wcat = jnp.concatenate([W_in, W_out], axis=1)
    wx = wcat[:D_FEAT].astype(jnp.bfloat16)
    we = wcat[D_FEAT:].astype(jnp.bfloat16)
    bcat = jnp.concatenate([b_in, b_out]).reshape(1, 2 * D_FEAT)

    sc_it_h = E_H // NS // K
    aggs = []
    hs = []
    for ph in range(2):
        e0 = ph * E_H
        g = _gather_h(x, lax.dynamic_slice_in_dim(col_p, e0, E_H))
        hs.append(_edge_mlp(
            g, lax.dynamic_slice_in_dim(ea_p, e0, E_H),
            lax.dynamic_slice_in_dim(dirf, e0, E_H), wx, we, bcat))
    for ph in range(2):
        tsel4 = tsel[:, ph * E_H:(ph + 1) * E_H].reshape(NC, NS, sc_it_h, K)
        hA, hB = hs[ph]
        agg = _scatter_h(hA, hB, tsel4)
        aggs.append(agg.reshape(NC * N_PAD, 2 * K))
    out = _node_mlp(aggs[0], aggs[1], W_node[:D_FEAT], W_node[D_FEAT:],
                    b_node.reshape(1, D_FEAT))
    return out[:N_NODES]
